# baseline (speedup 1.0000x reference)
import functools
import jax
import jax.numpy as jnp
from jax import lax
from jax.experimental import pallas as pl
from jax.experimental.pallas import tpu as pltpu
from jax.experimental.pallas import tpu_sc as plsc

_NQ = 10000
_E = 160000
_D = 128
_L = 32
_ROW = 136
_QROW = 256
_QW = 200
_NWIN = _NQ // _QW
_C = 640
_NCK = _E // _C
_TMAX = _NCK + _NWIN
_CH = 128
_NCH = _E // _CH
_NW = 32
_BQ = 1000


def _qtab_body(qf_ref, wq_ref, out_ref):
    q = jnp.dot(qf_ref[...], wq_ref[...], preferred_element_type=jnp.float32)
    out_ref[...] = q.astype(jnp.bfloat16)


def _gather_body(srcf, srcx, esrc, gf, gx,
                 idx0, idx1, fb0, fb1, xb0, xb1, sf0, sf1, sx0, sx1):
    c = lax.axis_index("c")
    s = lax.axis_index("s")
    wid = s * 2 + c
    nloop = (_NCH + _NW - 1) // _NW
    idx = (idx0, idx1)
    fb = (fb0, fb1)
    xb = (xb0, xb1)
    sf = (sf0, sf1)
    sx = (sx0, sx1)

    def issue(i, slot):
        chunk = jnp.minimum(wid + i * _NW, _NCH - 1)
        pltpu.sync_copy(esrc.at[pl.ds(chunk * _CH, _CH)], idx[slot])
        d1 = pltpu.async_copy(srcf.at[idx[slot]], fb[slot], sf[slot])
        d2 = pltpu.async_copy(srcx.at[idx[slot]], xb[slot], sx[slot])
        return chunk, d1, d2

    pend = issue(0, 0)
    for i in range(nloop):
        slot = i % 2
        nxt = issue(i + 1, 1 - slot) if i + 1 < nloop else None
        chunk, d1, d2 = pend
        d1.wait()
        d2.wait()
        pltpu.sync_copy(fb[slot], gf.at[pl.ds(chunk * _CH, _CH)])
        pltpu.sync_copy(xb[slot], gx.at[pl.ds(chunk * _CH, _CH)])
        pend = nxt


def _chunk_body(win_ref, cb_ref, msk_ref, gf_ref, gx_ref, qtab_ref, qx_ref,
                ed_ref, wpre_ref, bpre_ref, cen_ref, wgate_ref, wsh0_ref,
                wshp_ref, wk_ref, wv_ref, a_ref, s_ref, out_ref, *, gamma):
    bf16 = jnp.bfloat16
    t = pl.program_id(0)
    w = win_ref[t]
    edc = ed_ref[...].reshape(1, _C)
    rowid = w * _QW + lax.broadcasted_iota(jnp.int32, (_QW, 1), 0)
    oht = (edc == rowid).astype(jnp.float32)
    ohtb = oht.astype(bf16)
    qrow = lax.dot_general(ohtb, qtab_ref[...], (((0,), (0,)), ((), ())),
                           preferred_element_type=jnp.float32)
    qx = lax.dot_general(oht, qx_ref[...], (((0,), (0,)), ((), ())),
                         preferred_element_type=jnp.float32)
    gf = gf_ref[...]
    gx = gx_ref[:, 0:16]
    rel = gx - qx
    l2 = jnp.sum(rel * rel, axis=1, keepdims=True) + 1e-12
    length = jnp.sqrt(l2)
    dd = length - cen_ref[...]
    rbf = jnp.exp((-gamma) * dd * dd)
    pre = jnp.dot(rbf, wpre_ref[...], preferred_element_type=jnp.float32) + bpre_ref[...]
    scal = pre * jax.nn.sigmoid(pre)
    gate = jnp.dot(scal.astype(bf16), wgate_ref[...],
                   preferred_element_type=jnp.float32)
    inv = 1.7320508075688772 / length
    shw = wsh0_ref[...] + inv * jnp.dot(rel, wshp_ref[...],
                                        preferred_element_type=jnp.float32)
    m = (gf * gate + shw).astype(bf16)
    k = jnp.dot(m, wk_ref[...], preferred_element_type=jnp.float32)
    v = jnp.dot(m, wv_ref[...], preferred_element_type=jnp.float32)
    feat = qrow + k
    feat = jnp.where(feat >= 0, feat, 0.2 * feat)
    logits = jnp.dot(feat, a_ref[...], preferred_element_type=jnp.float32)
    ex = jnp.exp(logits)
    exb = jnp.dot(ex, s_ref[...], preferred_element_type=jnp.float32)
    exv = (v * exb).astype(bf16)

    mskf = msk_ref[t].astype(jnp.float32)
    pv = jnp.dot(ohtb, exv, preferred_element_type=jnp.float32) * mskf
    pe = jnp.dot(oht, ex, preferred_element_type=jnp.float32) * mskf
    partial = jnp.concatenate([pv, pe], axis=1)

    tp = jnp.maximum(t - 1, 0)
    first = jnp.logical_or(t == 0, win_ref[t] != win_ref[tp])

    @pl.when(first)
    def _():
        out_ref[...] = partial

    @pl.when(jnp.logical_not(first))
    def _():
        out_ref[...] = out_ref[...] + partial


def _final_body(a1_ref, a2_ref, qf_ref, wo_ref, s_ref, out_ref):
    a1 = a1_ref[...]
    a2 = a2_ref[...]
    numer = a1[:, 0:128] + a2[:, 0:128]
    den = a1[:, 128:136] + a2[:, 128:136]
    denb = jnp.dot(den, s_ref[...], preferred_element_type=jnp.float32) + 1e-9
    agg = numer / denb
    out_ref[...] = jnp.dot(agg, wo_ref[...],
                           preferred_element_type=jnp.float32) + qf_ref[...]


def _sc_mesh():
    return plsc.VectorSubcoreMesh(core_axis_name="c", subcore_axis_name="s",
                                  num_cores=2, num_subcores=16)


def _schedule(edst):
    qs = jnp.arange(0, _NQ + 1, _QW, dtype=jnp.int32)
    s = jnp.searchsorted(edst, qs).astype(jnp.int32)
    lo = s[:-1] // _C
    hi = -(-s[1:] // _C)
    n = jnp.maximum(hi - lo, 1)
    first = jnp.concatenate([jnp.zeros((1,), jnp.int32),
                             jnp.cumsum(n)[:-1].astype(jnp.int32)])
    win = jnp.repeat(jnp.arange(_NWIN, dtype=jnp.int32), n,
                     total_repeat_length=_TMAX)
    j = jnp.arange(_TMAX, dtype=jnp.int32) - first[win]
    msk = (j < n[win]).astype(jnp.int32)
    cb = jnp.clip(lo[win] + j, 0, _NCK - 1).astype(jnp.int32)
    return win, cb, msk


def kernel(query_x, query_f, src_x1, src_f1, src_x2, src_f2,
           edge_src1, edge_dst1, edge_src2, edge_dst2,
           Wpre1, bpre1, Wpre2, bpre2, W_gate, W_sh, Wq, Wk, Wv, Wo, a_vec):
    f32 = jnp.float32
    qx16 = jnp.pad(query_x, ((0, 0), (0, 13)))
    sx1 = jnp.pad(src_x1, ((0, 0), (0, _D - 3)))
    sx2 = jnp.pad(src_x2, ((0, 0), (0, _D - 3)))
    ed3d1 = edge_dst1.reshape(_NCK, 1, _C)
    ed3d2 = edge_dst2.reshape(_NCK, 1, _C)
    wsh0 = W_sh[0:1, :]
    wshp = jnp.pad(W_sh[1:4, :], ((0, 13), (0, 0)))
    amat = (a_vec[:, :, None] * jnp.eye(4, dtype=f32)[:, None, :]).reshape(_D, 4)
    amat = jnp.pad(amat, ((0, 0), (0, 4)))
    smat = jnp.pad(jnp.repeat(jnp.eye(4, dtype=f32), 32, axis=1),
                   ((0, 4), (0, 0)))
    cen1 = jnp.linspace(0.0, 0.5, _L, dtype=f32).reshape(1, _L)
    cen2 = jnp.linspace(0.0, 1.0, _L, dtype=f32).reshape(1, _L)
    g1 = (_L / 0.5) ** 2
    g2 = (_L / 1.0) ** 2
    bp1 = bpre1.reshape(1, _L)
    bp2 = bpre2.reshape(1, _L)
    sched1 = _schedule(edge_dst1)
    sched2 = _schedule(edge_dst2)

    wkb = Wk.astype(jnp.bfloat16)
    wvb = Wv.astype(jnp.bfloat16)
    wgb = W_gate.astype(jnp.bfloat16)

    qtab = pl.pallas_call(
        _qtab_body,
        grid=(_NQ // _BQ,),
        in_specs=[
            pl.BlockSpec((_BQ, _D), lambda i: (i, 0)),
            pl.BlockSpec((_D, _D), lambda i: (0, 0)),
        ],
        out_specs=pl.BlockSpec((_BQ, _D), lambda i: (i, 0)),
        out_shape=jax.ShapeDtypeStruct((_NQ, _D), jnp.bfloat16),
    )(query_f, Wq)

    gather = functools.partial(
        pl.kernel,
        _gather_body,
        out_type=[
            jax.ShapeDtypeStruct((_E, _D), f32),
            jax.ShapeDtypeStruct((_E, _D), f32),
        ],
        mesh=_sc_mesh(),
        scratch_types=[
            pltpu.VMEM((_CH,), jnp.int32),
            pltpu.VMEM((_CH,), jnp.int32),
            pltpu.VMEM((_CH, _D), f32),
            pltpu.VMEM((_CH, _D), f32),
            pltpu.VMEM((_CH, _D), f32),
            pltpu.VMEM((_CH, _D), f32),
            pltpu.SemaphoreType.DMA,
            pltpu.SemaphoreType.DMA,
            pltpu.SemaphoreType.DMA,
            pltpu.SemaphoreType.DMA,
        ],
    )()
    gf1, gx1 = gather(src_f1, sx1, edge_src1)
    gf2, gx2 = gather(src_f2, sx2, edge_src2)

    def p2_call(sched, gf, gx, ed3d, wpre, bp, cen, gamma):
        win, cb, msk = sched
        grid_spec = pltpu.PrefetchScalarGridSpec(
            num_scalar_prefetch=3,
            grid=(_TMAX,),
            in_specs=[
                pl.BlockSpec((_C, _D), lambda t, w, c, m: (c[t], 0)),
                pl.BlockSpec((_C, _D), lambda t, w, c, m: (c[t], 0)),
                pl.BlockSpec((_QW, _D), lambda t, w, c, m: (w[t], 0)),
                pl.BlockSpec((_QW, 16), lambda t, w, c, m: (w[t], 0)),
                pl.BlockSpec((1, 1, _C), lambda t, w, c, m: (c[t], 0, 0)),
                pl.BlockSpec((_L, _L), lambda t, w, c, m: (0, 0)),
                pl.BlockSpec((1, _L), lambda t, w, c, m: (0, 0)),
                pl.BlockSpec((1, _L), lambda t, w, c, m: (0, 0)),
                pl.BlockSpec((_L, _D), lambda t, w, c, m: (0, 0)),
                pl.BlockSpec((1, _D), lambda t, w, c, m: (0, 0)),
                pl.BlockSpec((16, _D), lambda t, w, c, m: (0, 0)),
                pl.BlockSpec((_D, _D), lambda t, w, c, m: (0, 0)),
                pl.BlockSpec((_D, _D), lambda t, w, c, m: (0, 0)),
                pl.BlockSpec((_D, 8), lambda t, w, c, m: (0, 0)),
                pl.BlockSpec((8, _D), lambda t, w, c, m: (0, 0)),
            ],
            out_specs=pl.BlockSpec((_QW, _ROW), lambda t, w, c, m: (w[t], 0)),
        )
        return pl.pallas_call(
            functools.partial(_chunk_body, gamma=gamma),
            grid_spec=grid_spec,
            out_shape=jax.ShapeDtypeStruct((_NQ, _ROW), f32),
        )(win, cb, msk, gf, gx, qtab, qx16, ed3d, wpre, bp, cen, wgb, wsh0,
          wshp, wkb, wvb, amat, smat)

    acc1 = p2_call(sched1, gf1, gx1, ed3d1, Wpre1, bp1, cen1, g1)
    acc2 = p2_call(sched2, gf2, gx2, ed3d2, Wpre2, bp2, cen2, g2)

    out = pl.pallas_call(
        _final_body,
        grid=(_NQ // _BQ,),
        in_specs=[
            pl.BlockSpec((_BQ, _ROW), lambda i: (i, 0)),
            pl.BlockSpec((_BQ, _ROW), lambda i: (i, 0)),
            pl.BlockSpec((_BQ, _D), lambda i: (i, 0)),
            pl.BlockSpec((_D, _D), lambda i: (0, 0)),
            pl.BlockSpec((8, _D), lambda i: (0, 0)),
        ],
        out_specs=pl.BlockSpec((_BQ, _D), lambda i: (i, 0)),
        out_shape=jax.ShapeDtypeStruct((_NQ, _D), f32),
    )(acc1, acc2, query_f, Wo, smat)
    return out


# speedup vs baseline: 1.0486x; 1.0486x over previous
import functools
import jax
import jax.numpy as jnp
from jax import lax
from jax.experimental import pallas as pl
from jax.experimental.pallas import tpu as pltpu
from jax.experimental.pallas import tpu_sc as plsc

_NQ = 10000
_E = 160000
_D = 128
_L = 32
_ROW = 136
_QROW = 256
_QW = 200
_NWIN = _NQ // _QW
_C = 640
_NCK = _E // _C
_TMAX = _NCK + _NWIN
_CH = 128
_NCH = _E // _CH
_NW = 32
_BQ = 1000


def _qtab_body(qf_ref, wq_ref, out_ref):
    out_ref[...] = jnp.dot(qf_ref[...], wq_ref[...],
                           preferred_element_type=jnp.float32)


def _gather_body(srcf, srcx, esrc, gf, gx,
                 idx0, idx1, fb0, fb1, xb0, xb1, sf0, sf1, sx0, sx1):
    c = lax.axis_index("c")
    s = lax.axis_index("s")
    wid = s * 2 + c
    nloop = (_NCH + _NW - 1) // _NW
    idx = (idx0, idx1)
    fb = (fb0, fb1)
    xb = (xb0, xb1)
    sf = (sf0, sf1)
    sx = (sx0, sx1)

    def issue(i, slot):
        chunk = jnp.minimum(wid + i * _NW, _NCH - 1)
        pltpu.sync_copy(esrc.at[pl.ds(chunk * _CH, _CH)], idx[slot])
        d1 = pltpu.async_copy(srcf.at[idx[slot]], fb[slot], sf[slot])
        d2 = pltpu.async_copy(srcx.at[idx[slot]], xb[slot], sx[slot])
        return chunk, d1, d2

    pend = issue(0, 0)
    for i in range(nloop):
        slot = i % 2
        nxt = issue(i + 1, 1 - slot) if i + 1 < nloop else None
        chunk, d1, d2 = pend
        d1.wait()
        d2.wait()
        pltpu.sync_copy(fb[slot], gf.at[pl.ds(chunk * _CH, _CH)])
        pltpu.sync_copy(xb[slot], gx.at[pl.ds(chunk * _CH, _CH)])
        pend = nxt


def _chunk_body(win_ref, cb_ref, msk_ref, gf_ref, gx_ref, qtab_ref, qx_ref,
                ed_ref, wpre_ref, bpre_ref, cen_ref, wgate_ref, wsh0_ref,
                wshp_ref, wk_ref, wv_ref, a_ref, s_ref, out_ref, *, gamma):
    t = pl.program_id(0)
    w = win_ref[t]
    edc = ed_ref[...].reshape(1, _C)
    rowid = w * _QW + lax.broadcasted_iota(jnp.int32, (_QW, 1), 0)
    oht = (edc == rowid).astype(jnp.float32)
    qrow = lax.dot_general(oht, qtab_ref[...], (((0,), (0,)), ((), ())),
                           preferred_element_type=jnp.float32)
    qx = lax.dot_general(oht, qx_ref[...], (((0,), (0,)), ((), ())),
                         preferred_element_type=jnp.float32)
    gf = gf_ref[...]
    gx = gx_ref[:, 0:16]
    rel = gx - qx
    l2 = jnp.sum(rel * rel, axis=1, keepdims=True) + 1e-12
    length = jnp.sqrt(l2)
    dd = length - cen_ref[...]
    rbf = jnp.exp((-gamma) * dd * dd)
    pre = jnp.dot(rbf, wpre_ref[...], preferred_element_type=jnp.float32) + bpre_ref[...]
    scal = pre * jax.nn.sigmoid(pre)
    gate = jnp.dot(scal, wgate_ref[...], preferred_element_type=jnp.float32)
    inv = 1.7320508075688772 / length
    shw = wsh0_ref[...] + inv * jnp.dot(rel, wshp_ref[...],
                                        preferred_element_type=jnp.float32)
    m = gf * gate + shw
    k = jnp.dot(m, wk_ref[...], preferred_element_type=jnp.float32)
    v = jnp.dot(m, wv_ref[...], preferred_element_type=jnp.float32)
    feat = qrow + k
    feat = jnp.where(feat >= 0, feat, 0.2 * feat)
    logits = jnp.dot(feat, a_ref[...], preferred_element_type=jnp.float32)
    ex = jnp.exp(logits)
    exb = jnp.dot(ex, s_ref[...], preferred_element_type=jnp.float32)
    exv = v * exb

    mskf = msk_ref[t].astype(jnp.float32)
    pv = jnp.dot(oht, exv, preferred_element_type=jnp.float32) * mskf
    pe = jnp.dot(oht, ex, preferred_element_type=jnp.float32) * mskf
    partial = jnp.concatenate([pv, pe], axis=1)

    tp = jnp.maximum(t - 1, 0)
    first = jnp.logical_or(t == 0, win_ref[t] != win_ref[tp])

    @pl.when(first)
    def _():
        out_ref[...] = partial

    @pl.when(jnp.logical_not(first))
    def _():
        out_ref[...] = out_ref[...] + partial


def _final_body(a1_ref, a2_ref, qf_ref, wo_ref, s_ref, out_ref):
    a1 = a1_ref[...]
    a2 = a2_ref[...]
    numer = a1[:, 0:128] + a2[:, 0:128]
    den = a1[:, 128:136] + a2[:, 128:136]
    denb = jnp.dot(den, s_ref[...], preferred_element_type=jnp.float32) + 1e-9
    agg = numer / denb
    out_ref[...] = jnp.dot(agg, wo_ref[...],
                           preferred_element_type=jnp.float32) + qf_ref[...]


def _sc_mesh():
    return plsc.VectorSubcoreMesh(core_axis_name="c", subcore_axis_name="s",
                                  num_cores=2, num_subcores=16)


def _schedule(edst):
    qs = jnp.arange(0, _NQ + 1, _QW, dtype=jnp.int32)
    s = jnp.searchsorted(edst, qs).astype(jnp.int32)
    lo = s[:-1] // _C
    hi = -(-s[1:] // _C)
    n = jnp.maximum(hi - lo, 1)
    first = jnp.concatenate([jnp.zeros((1,), jnp.int32),
                             jnp.cumsum(n)[:-1].astype(jnp.int32)])
    win = jnp.repeat(jnp.arange(_NWIN, dtype=jnp.int32), n,
                     total_repeat_length=_TMAX)
    j = jnp.arange(_TMAX, dtype=jnp.int32) - first[win]
    msk = (j < n[win]).astype(jnp.int32)
    cb = jnp.clip(lo[win] + j, 0, _NCK - 1).astype(jnp.int32)
    return win, cb, msk


def kernel(query_x, query_f, src_x1, src_f1, src_x2, src_f2,
           edge_src1, edge_dst1, edge_src2, edge_dst2,
           Wpre1, bpre1, Wpre2, bpre2, W_gate, W_sh, Wq, Wk, Wv, Wo, a_vec):
    f32 = jnp.float32
    qx16 = jnp.pad(query_x, ((0, 0), (0, 13)))
    sx1 = jnp.pad(src_x1, ((0, 0), (0, _D - 3)))
    sx2 = jnp.pad(src_x2, ((0, 0), (0, _D - 3)))
    ed3d1 = edge_dst1.reshape(_NCK, 1, _C)
    ed3d2 = edge_dst2.reshape(_NCK, 1, _C)
    wsh0 = W_sh[0:1, :]
    wshp = jnp.pad(W_sh[1:4, :], ((0, 13), (0, 0)))
    amat = (a_vec[:, :, None] * jnp.eye(4, dtype=f32)[:, None, :]).reshape(_D, 4)
    amat = jnp.pad(amat, ((0, 0), (0, 4)))
    smat = jnp.pad(jnp.repeat(jnp.eye(4, dtype=f32), 32, axis=1),
                   ((0, 4), (0, 0)))
    cen1 = jnp.linspace(0.0, 0.5, _L, dtype=f32).reshape(1, _L)
    cen2 = jnp.linspace(0.0, 1.0, _L, dtype=f32).reshape(1, _L)
    g1 = (_L / 0.5) ** 2
    g2 = (_L / 1.0) ** 2
    bp1 = bpre1.reshape(1, _L)
    bp2 = bpre2.reshape(1, _L)
    sched1 = _schedule(edge_dst1)
    sched2 = _schedule(edge_dst2)

    qtab = pl.pallas_call(
        _qtab_body,
        grid=(_NQ // _BQ,),
        in_specs=[
            pl.BlockSpec((_BQ, _D), lambda i: (i, 0)),
            pl.BlockSpec((_D, _D), lambda i: (0, 0)),
        ],
        out_specs=pl.BlockSpec((_BQ, _D), lambda i: (i, 0)),
        out_shape=jax.ShapeDtypeStruct((_NQ, _D), f32),
    )(query_f, Wq)

    gather = functools.partial(
        pl.kernel,
        _gather_body,
        out_type=[
            jax.ShapeDtypeStruct((_E, _D), f32),
            jax.ShapeDtypeStruct((_E, _D), f32),
        ],
        mesh=_sc_mesh(),
        scratch_types=[
            pltpu.VMEM((_CH,), jnp.int32),
            pltpu.VMEM((_CH,), jnp.int32),
            pltpu.VMEM((_CH, _D), f32),
            pltpu.VMEM((_CH, _D), f32),
            pltpu.VMEM((_CH, _D), f32),
            pltpu.VMEM((_CH, _D), f32),
            pltpu.SemaphoreType.DMA,
            pltpu.SemaphoreType.DMA,
            pltpu.SemaphoreType.DMA,
            pltpu.SemaphoreType.DMA,
        ],
    )()
    gf1, gx1 = gather(src_f1, sx1, edge_src1)
    gf2, gx2 = gather(src_f2, sx2, edge_src2)

    def p2_call(sched, gf, gx, ed3d, wpre, bp, cen, gamma):
        win, cb, msk = sched
        grid_spec = pltpu.PrefetchScalarGridSpec(
            num_scalar_prefetch=3,
            grid=(_TMAX,),
            in_specs=[
                pl.BlockSpec((_C, _D), lambda t, w, c, m: (c[t], 0)),
                pl.BlockSpec((_C, _D), lambda t, w, c, m: (c[t], 0)),
                pl.BlockSpec((_QW, _D), lambda t, w, c, m: (w[t], 0)),
                pl.BlockSpec((_QW, 16), lambda t, w, c, m: (w[t], 0)),
                pl.BlockSpec((1, 1, _C), lambda t, w, c, m: (c[t], 0, 0)),
                pl.BlockSpec((_L, _L), lambda t, w, c, m: (0, 0)),
                pl.BlockSpec((1, _L), lambda t, w, c, m: (0, 0)),
                pl.BlockSpec((1, _L), lambda t, w, c, m: (0, 0)),
                pl.BlockSpec((_L, _D), lambda t, w, c, m: (0, 0)),
                pl.BlockSpec((1, _D), lambda t, w, c, m: (0, 0)),
                pl.BlockSpec((16, _D), lambda t, w, c, m: (0, 0)),
                pl.BlockSpec((_D, _D), lambda t, w, c, m: (0, 0)),
                pl.BlockSpec((_D, _D), lambda t, w, c, m: (0, 0)),
                pl.BlockSpec((_D, 8), lambda t, w, c, m: (0, 0)),
                pl.BlockSpec((8, _D), lambda t, w, c, m: (0, 0)),
            ],
            out_specs=pl.BlockSpec((_QW, _ROW), lambda t, w, c, m: (w[t], 0)),
        )
        return pl.pallas_call(
            functools.partial(_chunk_body, gamma=gamma),
            grid_spec=grid_spec,
            out_shape=jax.ShapeDtypeStruct((_NQ, _ROW), f32),
        )(win, cb, msk, gf, gx, qtab, qx16, ed3d, wpre, bp, cen, W_gate, wsh0,
          wshp, Wk, Wv, amat, smat)

    acc1 = p2_call(sched1, gf1, gx1, ed3d1, Wpre1, bp1, cen1, g1)
    acc2 = p2_call(sched2, gf2, gx2, ed3d2, Wpre2, bp2, cen2, g2)

    out = pl.pallas_call(
        _final_body,
        grid=(_NQ // _BQ,),
        in_specs=[
            pl.BlockSpec((_BQ, _ROW), lambda i: (i, 0)),
            pl.BlockSpec((_BQ, _ROW), lambda i: (i, 0)),
            pl.BlockSpec((_BQ, _D), lambda i: (i, 0)),
            pl.BlockSpec((_D, _D), lambda i: (0, 0)),
            pl.BlockSpec((8, _D), lambda i: (0, 0)),
        ],
        out_specs=pl.BlockSpec((_BQ, _D), lambda i: (i, 0)),
        out_shape=jax.ShapeDtypeStruct((_NQ, _D), f32),
    )(acc1, acc2, query_f, Wo, smat)
    return out


# speedup vs baseline: 1.1060x; 1.0547x over previous
import functools
import jax
import jax.numpy as jnp
from jax import lax
from jax.experimental import pallas as pl
from jax.experimental.pallas import tpu as pltpu
from jax.experimental.pallas import tpu_sc as plsc

_NQ = 10000
_E = 160000
_D = 128
_L = 32
_ROW = 136
_QROW = 256
_QW = 200
_NWIN = _NQ // _QW
_C = 640
_NCK = _E // _C
_TMAX = _NCK + _NWIN
_CH = 128
_NCH = _E // _CH
_NW = 32
_BQ = 1000


def _qtab_body(qf_ref, qx_ref, wq_ref, out_ref):
    q = jnp.dot(qf_ref[...], wq_ref[...], preferred_element_type=jnp.float32)
    out_ref[...] = jnp.concatenate(
        [q, qx_ref[...], jnp.zeros((q.shape[0], _QROW - _D - 16), jnp.float32)],
        axis=1)


def _gather_body(srcf, srcx, esrc, gf, gx,
                 idx0, idx1, fb0, fb1, xb0, xb1, sf0, sf1, sx0, sx1):
    c = lax.axis_index("c")
    s = lax.axis_index("s")
    wid = s * 2 + c
    nloop = (_NCH + _NW - 1) // _NW
    idx = (idx0, idx1)
    fb = (fb0, fb1)
    xb = (xb0, xb1)
    sf = (sf0, sf1)
    sx = (sx0, sx1)

    def issue(i, slot):
        chunk = wid + i * _NW
        if i * _NW >= _NW:
            chunk = jnp.where(chunk < _NCH, chunk, chunk - _NW)
        pltpu.sync_copy(esrc.at[pl.ds(chunk * _CH, _CH)], idx[slot])
        d1 = pltpu.async_copy(srcf.at[idx[slot]], fb[slot], sf[slot])
        d2 = pltpu.async_copy(srcx.at[idx[slot]], xb[slot], sx[slot])
        return chunk, d1, d2

    pend = issue(0, 0)
    for i in range(nloop):
        slot = i % 2
        nxt = issue(i + 1, 1 - slot) if i + 1 < nloop else None
        chunk, d1, d2 = pend
        d1.wait()
        d2.wait()
        pltpu.sync_copy(fb[slot], gf.at[pl.ds(chunk * _CH, _CH)])
        pltpu.sync_copy(xb[slot], gx.at[pl.ds(chunk * _CH, _CH)])
        pend = nxt


def _chunk_body(win_ref, cb_ref, msk_ref, gf_ref, gx_ref, qtab_ref,
                ed_ref, wpre_ref, bpre_ref, cen_ref, wgate_ref, wsh0_ref,
                wshp_ref, wk_ref, wv_ref, a_ref, s_ref, out_ref, *, gamma):
    t = pl.program_id(0)
    w = win_ref[t]
    edc = ed_ref[...].reshape(1, _C)
    rowid = w * _QW + lax.broadcasted_iota(jnp.int32, (_QW, 1), 0)
    oht = (edc == rowid).astype(jnp.float32)
    gqc = lax.dot_general(oht, qtab_ref[...], (((0,), (0,)), ((), ())),
                          preferred_element_type=jnp.float32)
    qrow = gqc[:, 0:128]
    qx = gqc[:, 128:144]
    gf = gf_ref[...]
    gx = gx_ref[:, 0:16]
    rel = gx - qx
    l2 = jnp.sum(rel * rel, axis=1, keepdims=True) + 1e-12
    length = jnp.sqrt(l2)
    dd = length - cen_ref[...]
    rbf = jnp.exp((-gamma) * dd * dd)
    pre = jnp.dot(rbf, wpre_ref[...], preferred_element_type=jnp.float32) + bpre_ref[...]
    scal = pre * jax.nn.sigmoid(pre)
    gate = jnp.dot(scal, wgate_ref[...], preferred_element_type=jnp.float32)
    inv = 1.7320508075688772 / length
    shw = wsh0_ref[...] + inv * jnp.dot(rel, wshp_ref[...],
                                        preferred_element_type=jnp.float32)
    m = gf * gate + shw
    k = jnp.dot(m, wk_ref[...], preferred_element_type=jnp.float32)
    v = jnp.dot(m, wv_ref[...], preferred_element_type=jnp.float32)
    feat = qrow + k
    feat = jnp.where(feat >= 0, feat, 0.2 * feat)
    logits = jnp.dot(feat, a_ref[...], preferred_element_type=jnp.float32)
    ex = jnp.exp(logits)
    exb = jnp.dot(ex, s_ref[...], preferred_element_type=jnp.float32)
    exv = v * exb

    mskf = msk_ref[t].astype(jnp.float32)
    pv = jnp.dot(oht, exv, preferred_element_type=jnp.float32) * mskf
    pe = jnp.dot(oht, ex, preferred_element_type=jnp.float32) * mskf
    partial = jnp.concatenate([pv, pe], axis=1)

    tp = jnp.maximum(t - 1, 0)
    first = jnp.logical_or(t == 0, win_ref[t] != win_ref[tp])

    @pl.when(first)
    def _():
        out_ref[...] = partial

    @pl.when(jnp.logical_not(first))
    def _():
        out_ref[...] = out_ref[...] + partial


def _final_body(a1_ref, a2_ref, qf_ref, wo_ref, s_ref, out_ref):
    a1 = a1_ref[...]
    a2 = a2_ref[...]
    numer = a1[:, 0:128] + a2[:, 0:128]
    den = a1[:, 128:136] + a2[:, 128:136]
    denb = jnp.dot(den, s_ref[...], preferred_element_type=jnp.float32) + 1e-9
    agg = numer / denb
    out_ref[...] = jnp.dot(agg, wo_ref[...],
                           preferred_element_type=jnp.float32) + qf_ref[...]


def _sc_mesh():
    return plsc.VectorSubcoreMesh(core_axis_name="c", subcore_axis_name="s",
                                  num_cores=2, num_subcores=16)


def _schedule(edst):
    qs = jnp.arange(0, _NQ + 1, _QW, dtype=jnp.int32)
    s = jnp.searchsorted(edst, qs).astype(jnp.int32)
    lo = s[:-1] // _C
    hi = -(-s[1:] // _C)
    n = jnp.maximum(hi - lo, 1)
    first = jnp.concatenate([jnp.zeros((1,), jnp.int32),
                             jnp.cumsum(n)[:-1].astype(jnp.int32)])
    win = jnp.repeat(jnp.arange(_NWIN, dtype=jnp.int32), n,
                     total_repeat_length=_TMAX)
    j = jnp.arange(_TMAX, dtype=jnp.int32) - first[win]
    msk = (j < n[win]).astype(jnp.int32)
    cb = jnp.clip(lo[win] + j, 0, _NCK - 1).astype(jnp.int32)
    return win, cb, msk


def kernel(query_x, query_f, src_x1, src_f1, src_x2, src_f2,
           edge_src1, edge_dst1, edge_src2, edge_dst2,
           Wpre1, bpre1, Wpre2, bpre2, W_gate, W_sh, Wq, Wk, Wv, Wo, a_vec):
    f32 = jnp.float32
    qx16 = jnp.pad(query_x, ((0, 0), (0, 13)))
    sx1 = jnp.pad(src_x1, ((0, 0), (0, _D - 3)))
    sx2 = jnp.pad(src_x2, ((0, 0), (0, _D - 3)))
    ed3d1 = edge_dst1.reshape(_NCK, 1, _C)
    ed3d2 = edge_dst2.reshape(_NCK, 1, _C)
    wsh0 = W_sh[0:1, :]
    wshp = jnp.pad(W_sh[1:4, :], ((0, 13), (0, 0)))
    amat = (a_vec[:, :, None] * jnp.eye(4, dtype=f32)[:, None, :]).reshape(_D, 4)
    amat = jnp.pad(amat, ((0, 0), (0, 4)))
    smat = jnp.pad(jnp.repeat(jnp.eye(4, dtype=f32), 32, axis=1),
                   ((0, 4), (0, 0)))
    cen1 = jnp.linspace(0.0, 0.5, _L, dtype=f32).reshape(1, _L)
    cen2 = jnp.linspace(0.0, 1.0, _L, dtype=f32).reshape(1, _L)
    g1 = (_L / 0.5) ** 2
    g2 = (_L / 1.0) ** 2
    bp1 = bpre1.reshape(1, _L)
    bp2 = bpre2.reshape(1, _L)
    sched1 = _schedule(edge_dst1)
    sched2 = _schedule(edge_dst2)

    qtab = pl.pallas_call(
        _qtab_body,
        grid=(_NQ // _BQ,),
        in_specs=[
            pl.BlockSpec((_BQ, _D), lambda i: (i, 0)),
            pl.BlockSpec((_BQ, 16), lambda i: (i, 0)),
            pl.BlockSpec((_D, _D), lambda i: (0, 0)),
        ],
        out_specs=pl.BlockSpec((_BQ, _QROW), lambda i: (i, 0)),
        out_shape=jax.ShapeDtypeStruct((_NQ, _QROW), f32),
    )(query_f, qx16, Wq)

    gather = functools.partial(
        pl.kernel,
        _gather_body,
        out_type=[
            jax.ShapeDtypeStruct((_E, _D), f32),
            jax.ShapeDtypeStruct((_E, _D), f32),
        ],
        mesh=_sc_mesh(),
        scratch_types=[
            pltpu.VMEM((_CH,), jnp.int32),
            pltpu.VMEM((_CH,), jnp.int32),
            pltpu.VMEM((_CH, _D), f32),
            pltpu.VMEM((_CH, _D), f32),
            pltpu.VMEM((_CH, _D), f32),
            pltpu.VMEM((_CH, _D), f32),
            pltpu.SemaphoreType.DMA,
            pltpu.SemaphoreType.DMA,
            pltpu.SemaphoreType.DMA,
            pltpu.SemaphoreType.DMA,
        ],
    )()
    gf1, gx1 = gather(src_f1, sx1, edge_src1)
    gf2, gx2 = gather(src_f2, sx2, edge_src2)

    def p2_call(sched, gf, gx, ed3d, wpre, bp, cen, gamma):
        win, cb, msk = sched
        grid_spec = pltpu.PrefetchScalarGridSpec(
            num_scalar_prefetch=3,
            grid=(_TMAX,),
            in_specs=[
                pl.BlockSpec((_C, _D), lambda t, w, c, m: (c[t], 0)),
                pl.BlockSpec((_C, _D), lambda t, w, c, m: (c[t], 0)),
                pl.BlockSpec((_QW, _QROW), lambda t, w, c, m: (w[t], 0)),
                pl.BlockSpec((1, 1, _C), lambda t, w, c, m: (c[t], 0, 0)),
                pl.BlockSpec((_L, _L), lambda t, w, c, m: (0, 0)),
                pl.BlockSpec((1, _L), lambda t, w, c, m: (0, 0)),
                pl.BlockSpec((1, _L), lambda t, w, c, m: (0, 0)),
                pl.BlockSpec((_L, _D), lambda t, w, c, m: (0, 0)),
                pl.BlockSpec((1, _D), lambda t, w, c, m: (0, 0)),
                pl.BlockSpec((16, _D), lambda t, w, c, m: (0, 0)),
                pl.BlockSpec((_D, _D), lambda t, w, c, m: (0, 0)),
                pl.BlockSpec((_D, _D), lambda t, w, c, m: (0, 0)),
                pl.BlockSpec((_D, 8), lambda t, w, c, m: (0, 0)),
                pl.BlockSpec((8, _D), lambda t, w, c, m: (0, 0)),
            ],
            out_specs=pl.BlockSpec((_QW, _ROW), lambda t, w, c, m: (w[t], 0)),
        )
        return pl.pallas_call(
            functools.partial(_chunk_body, gamma=gamma),
            grid_spec=grid_spec,
            out_shape=jax.ShapeDtypeStruct((_NQ, _ROW), f32),
        )(win, cb, msk, gf, gx, qtab, ed3d, wpre, bp, cen, W_gate, wsh0,
          wshp, Wk, Wv, amat, smat)

    acc1 = p2_call(sched1, gf1, gx1, ed3d1, Wpre1, bp1, cen1, g1)
    acc2 = p2_call(sched2, gf2, gx2, ed3d2, Wpre2, bp2, cen2, g2)

    out = pl.pallas_call(
        _final_body,
        grid=(_NQ // _BQ,),
        in_specs=[
            pl.BlockSpec((_BQ, _ROW), lambda i: (i, 0)),
            pl.BlockSpec((_BQ, _ROW), lambda i: (i, 0)),
            pl.BlockSpec((_BQ, _D), lambda i: (i, 0)),
            pl.BlockSpec((_D, _D), lambda i: (0, 0)),
            pl.BlockSpec((8, _D), lambda i: (0, 0)),
        ],
        out_specs=pl.BlockSpec((_BQ, _D), lambda i: (i, 0)),
        out_shape=jax.ShapeDtypeStruct((_NQ, _D), f32),
    )(acc1, acc2, query_f, Wo, smat)
    return out


# speedup vs baseline: 1.7311x; 1.5652x over previous
import functools
import jax
import jax.numpy as jnp
from jax import lax
from jax.experimental import pallas as pl
from jax.experimental.pallas import tpu as pltpu
from jax.experimental.pallas import tpu_sc as plsc

_NQ = 10000
_E = 160000
_D = 128
_L = 32
_ROW = 136
_QROW = 256
_QW = 200
_NWIN = _NQ // _QW
_C = 1280
_NCK = _E // _C
_TMAX = _NCK + _NWIN
_CH = 128
_NCH = _E // _CH
_NW = 32
_BQ = 1000


def _qtab_body(qf_ref, qx_ref, wq_ref, out_ref):
    q = jnp.dot(qf_ref[...], wq_ref[...], preferred_element_type=jnp.float32)
    out_ref[...] = jnp.concatenate(
        [q, qx_ref[...], jnp.zeros((q.shape[0], _QROW - _D - 16), jnp.float32)],
        axis=1)


def _gather_body(srcf, srcx, esrc, gf, gx, idx_s, fbuf, xbuf, sem):
    c = lax.axis_index("c")
    s = lax.axis_index("s")
    wid = s * 2 + c
    nloop = (_NCH + _NW - 1) // _NW

    def body(i, carry):
        chunk = wid + i * _NW

        @pl.when(chunk < _NCH)
        def _():
            base = chunk * _CH
            pltpu.sync_copy(esrc.at[pl.ds(base, _CH)], idx_s)
            pltpu.async_copy(srcf.at[idx_s], fbuf, sem).wait()
            pltpu.async_copy(srcx.at[idx_s], xbuf, sem).wait()
            pltpu.sync_copy(fbuf, gf.at[pl.ds(base, _CH)])
            pltpu.sync_copy(xbuf, gx.at[pl.ds(base, _CH)])
        return carry

    lax.fori_loop(0, nloop, body, 0)


def _chunk_body(win_ref, cb_ref, msk_ref, gf_ref, gx_ref, qtab_ref,
                ed_ref, wpre_ref, bpre_ref, cen_ref, wgate_ref, wsh0_ref,
                wshp_ref, wk_ref, wv_ref, a_ref, s_ref, out_ref, *, gamma):
    t = pl.program_id(0)
    w = win_ref[t]
    edc = ed_ref[...].reshape(1, _C)
    rowid = w * _QW + lax.broadcasted_iota(jnp.int32, (_QW, 1), 0)
    oht = (edc == rowid).astype(jnp.float32)
    gqc = lax.dot_general(oht, qtab_ref[...], (((0,), (0,)), ((), ())),
                          preferred_element_type=jnp.float32)
    qrow = gqc[:, 0:128]
    qx = gqc[:, 128:144]
    gf = gf_ref[...]
    gx = gx_ref[:, 0:16]
    rel = gx - qx
    l2 = jnp.sum(rel * rel, axis=1, keepdims=True) + 1e-12
    length = jnp.sqrt(l2)
    dd = length - cen_ref[...]
    rbf = jnp.exp((-gamma) * dd * dd)
    pre = jnp.dot(rbf, wpre_ref[...], preferred_element_type=jnp.float32) + bpre_ref[...]
    scal = pre * jax.nn.sigmoid(pre)
    gate = jnp.dot(scal, wgate_ref[...], preferred_element_type=jnp.float32)
    inv = 1.7320508075688772 / length
    shw = wsh0_ref[...] + inv * jnp.dot(rel, wshp_ref[...],
                                        preferred_element_type=jnp.float32)
    m = gf * gate + shw
    k = jnp.dot(m, wk_ref[...], preferred_element_type=jnp.float32)
    v = jnp.dot(m, wv_ref[...], preferred_element_type=jnp.float32)
    feat = qrow + k
    feat = jnp.where(feat >= 0, feat, 0.2 * feat)
    logits = jnp.dot(feat, a_ref[...], preferred_element_type=jnp.float32)
    ex = jnp.exp(logits)
    exb = jnp.dot(ex, s_ref[...], preferred_element_type=jnp.float32)
    exv = v * exb

    mskf = msk_ref[t].astype(jnp.float32)
    pv = jnp.dot(oht, exv, preferred_element_type=jnp.float32) * mskf
    pe = jnp.dot(oht, ex, preferred_element_type=jnp.float32) * mskf
    partial = jnp.concatenate([pv, pe], axis=1)

    tp = jnp.maximum(t - 1, 0)
    first = jnp.logical_or(t == 0, win_ref[t] != win_ref[tp])

    @pl.when(first)
    def _():
        out_ref[...] = partial

    @pl.when(jnp.logical_not(first))
    def _():
        out_ref[...] = out_ref[...] + partial


def _final_body(a1_ref, a2_ref, qf_ref, wo_ref, s_ref, out_ref):
    a1 = a1_ref[...]
    a2 = a2_ref[...]
    numer = a1[:, 0:128] + a2[:, 0:128]
    den = a1[:, 128:136] + a2[:, 128:136]
    denb = jnp.dot(den, s_ref[...], preferred_element_type=jnp.float32) + 1e-9
    agg = numer / denb
    out_ref[...] = jnp.dot(agg, wo_ref[...],
                           preferred_element_type=jnp.float32) + qf_ref[...]


def _sc_mesh():
    return plsc.VectorSubcoreMesh(core_axis_name="c", subcore_axis_name="s",
                                  num_cores=2, num_subcores=16)


def _schedule(edst):
    qs = jnp.arange(0, _NQ + 1, _QW, dtype=jnp.int32)
    s = jnp.searchsorted(edst, qs).astype(jnp.int32)
    lo = s[:-1] // _C
    hi = -(-s[1:] // _C)
    n = jnp.maximum(hi - lo, 1)
    first = jnp.concatenate([jnp.zeros((1,), jnp.int32),
                             jnp.cumsum(n)[:-1].astype(jnp.int32)])
    win = jnp.repeat(jnp.arange(_NWIN, dtype=jnp.int32), n,
                     total_repeat_length=_TMAX)
    j = jnp.arange(_TMAX, dtype=jnp.int32) - first[win]
    msk = (j < n[win]).astype(jnp.int32)
    cb = jnp.clip(lo[win] + j, 0, _NCK - 1).astype(jnp.int32)
    return win, cb, msk


def kernel(query_x, query_f, src_x1, src_f1, src_x2, src_f2,
           edge_src1, edge_dst1, edge_src2, edge_dst2,
           Wpre1, bpre1, Wpre2, bpre2, W_gate, W_sh, Wq, Wk, Wv, Wo, a_vec):
    f32 = jnp.float32
    qx16 = jnp.pad(query_x, ((0, 0), (0, 13)))
    sx1 = jnp.pad(src_x1, ((0, 0), (0, _D - 3)))
    sx2 = jnp.pad(src_x2, ((0, 0), (0, _D - 3)))
    ed3d1 = edge_dst1.reshape(_NCK, 1, _C)
    ed3d2 = edge_dst2.reshape(_NCK, 1, _C)
    wsh0 = W_sh[0:1, :]
    wshp = jnp.pad(W_sh[1:4, :], ((0, 13), (0, 0)))
    amat = (a_vec[:, :, None] * jnp.eye(4, dtype=f32)[:, None, :]).reshape(_D, 4)
    amat = jnp.pad(amat, ((0, 0), (0, 4)))
    smat = jnp.pad(jnp.repeat(jnp.eye(4, dtype=f32), 32, axis=1),
                   ((0, 4), (0, 0)))
    cen1 = jnp.linspace(0.0, 0.5, _L, dtype=f32).reshape(1, _L)
    cen2 = jnp.linspace(0.0, 1.0, _L, dtype=f32).reshape(1, _L)
    g1 = (_L / 0.5) ** 2
    g2 = (_L / 1.0) ** 2
    bp1 = bpre1.reshape(1, _L)
    bp2 = bpre2.reshape(1, _L)
    sched1 = _schedule(edge_dst1)
    sched2 = _schedule(edge_dst2)

    qtab = pl.pallas_call(
        _qtab_body,
        grid=(_NQ // _BQ,),
        in_specs=[
            pl.BlockSpec((_BQ, _D), lambda i: (i, 0)),
            pl.BlockSpec((_BQ, 16), lambda i: (i, 0)),
            pl.BlockSpec((_D, _D), lambda i: (0, 0)),
        ],
        out_specs=pl.BlockSpec((_BQ, _QROW), lambda i: (i, 0)),
        out_shape=jax.ShapeDtypeStruct((_NQ, _QROW), f32),
    )(query_f, qx16, Wq)

    gather = functools.partial(
        pl.kernel,
        _gather_body,
        out_type=[
            jax.ShapeDtypeStruct((_E, _D), f32),
            jax.ShapeDtypeStruct((_E, _D), f32),
        ],
        mesh=_sc_mesh(),
        scratch_types=[
            pltpu.VMEM((_CH,), jnp.int32),
            pltpu.VMEM((_CH, _D), f32),
            pltpu.VMEM((_CH, _D), f32),
            pltpu.SemaphoreType.DMA,
        ],
    )()
    gf1, gx1 = gather(src_f1, sx1, edge_src1)
    gf2, gx2 = gather(src_f2, sx2, edge_src2)

    def p2_call(sched, gf, gx, ed3d, wpre, bp, cen, gamma):
        win, cb, msk = sched
        grid_spec = pltpu.PrefetchScalarGridSpec(
            num_scalar_prefetch=3,
            grid=(_TMAX,),
            in_specs=[
                pl.BlockSpec((_C, _D), lambda t, w, c, m: (c[t], 0)),
                pl.BlockSpec((_C, _D), lambda t, w, c, m: (c[t], 0)),
                pl.BlockSpec((_QW, _QROW), lambda t, w, c, m: (w[t], 0)),
                pl.BlockSpec((1, 1, _C), lambda t, w, c, m: (c[t], 0, 0)),
                pl.BlockSpec((_L, _L), lambda t, w, c, m: (0, 0)),
                pl.BlockSpec((1, _L), lambda t, w, c, m: (0, 0)),
                pl.BlockSpec((1, _L), lambda t, w, c, m: (0, 0)),
                pl.BlockSpec((_L, _D), lambda t, w, c, m: (0, 0)),
                pl.BlockSpec((1, _D), lambda t, w, c, m: (0, 0)),
                pl.BlockSpec((16, _D), lambda t, w, c, m: (0, 0)),
                pl.BlockSpec((_D, _D), lambda t, w, c, m: (0, 0)),
                pl.BlockSpec((_D, _D), lambda t, w, c, m: (0, 0)),
                pl.BlockSpec((_D, 8), lambda t, w, c, m: (0, 0)),
                pl.BlockSpec((8, _D), lambda t, w, c, m: (0, 0)),
            ],
            out_specs=pl.BlockSpec((_QW, _ROW), lambda t, w, c, m: (w[t], 0)),
        )
        return pl.pallas_call(
            functools.partial(_chunk_body, gamma=gamma),
            grid_spec=grid_spec,
            out_shape=jax.ShapeDtypeStruct((_NQ, _ROW), f32),
        )(win, cb, msk, gf, gx, qtab, ed3d, wpre, bp, cen, W_gate, wsh0,
          wshp, Wk, Wv, amat, smat)

    acc1 = p2_call(sched1, gf1, gx1, ed3d1, Wpre1, bp1, cen1, g1)
    acc2 = p2_call(sched2, gf2, gx2, ed3d2, Wpre2, bp2, cen2, g2)

    out = pl.pallas_call(
        _final_body,
        grid=(_NQ // _BQ,),
        in_specs=[
            pl.BlockSpec((_BQ, _ROW), lambda i: (i, 0)),
            pl.BlockSpec((_BQ, _ROW), lambda i: (i, 0)),
            pl.BlockSpec((_BQ, _D), lambda i: (i, 0)),
            pl.BlockSpec((_D, _D), lambda i: (0, 0)),
            pl.BlockSpec((8, _D), lambda i: (0, 0)),
        ],
        out_specs=pl.BlockSpec((_BQ, _D), lambda i: (i, 0)),
        out_shape=jax.ShapeDtypeStruct((_NQ, _D), f32),
    )(acc1, acc2, query_f, Wo, smat)
    return out
